# baseline (speedup 1.0000x reference)
import functools

import jax
import jax.numpy as jnp
from jax import lax
from jax.experimental import pallas as pl
from jax.experimental.pallas import tpu as pltpu
from jax.experimental.pallas import tpu_sc as plsc

BATCH = 16384
NUM_FIELDS = 26
EMBED_DIM = 16
FIELD_DIM = 40000
TOTAL = BATCH * NUM_FIELDS
SUM_FIELD = NUM_FIELDS * FIELD_DIM

NC, NS, LANES = 2, 16, 16
NW = NC * NS

NPAIR = SUM_FIELD // 128
PAIRS_W = -(-NPAIR // NW)

PER_W = TOTAL // NW
GROUPS_W = PER_W // 128

_MESH = dict(core_axis_name="c", subcore_axis_name="s")
_PARAMS = pltpu.CompilerParams(
    use_tc_tiling_on_sc=False,
    needs_layout_passes=False,
    disable_bounds_checks=True,
)


def _transpose_pair(src, dst, iota):
    for b in range(128):
        col = jnp.full((LANES,), b, jnp.int32)
        dst[b, :] = plsc.load_gather(src, [iota, col])


def _convert_table():
    @functools.partial(
        pl.kernel,
        out_type=jax.ShapeDtypeStruct((SUM_FIELD, EMBED_DIM), jnp.float32),
        mesh=plsc.VectorSubcoreMesh(**_MESH),
        compiler_params=_PARAMS,
        scratch_types=[
            pltpu.VMEM((16, 128), jnp.float32),
            pltpu.VMEM((16, 128), jnp.float32),
            pltpu.VMEM((128, EMBED_DIM), jnp.float32),
            pltpu.VMEM((128, EMBED_DIM), jnp.float32),
            pltpu.SemaphoreType.DMA,
            pltpu.SemaphoreType.DMA,
            pltpu.SemaphoreType.DMA,
            pltpu.SemaphoreType.DMA,
        ],
    )
    def k(t5, tbl, in_a, in_b, out_a, out_b, sg_a, sg_b, sw_a, sw_b):
        wid = lax.axis_index("s") * NC + lax.axis_index("c")
        start = wid * PAIRS_W
        iota = lax.iota(jnp.int32, LANES)

        def jj(t):
            return jnp.minimum(start + t, NPAIR - 1)

        def in_cps(t, buf, sem):
            j = jj(t)
            return (
                pltpu.make_async_copy(t5.at[0, j], buf.at[pl.ds(0, 8)], sem),
                pltpu.make_async_copy(t5.at[1, j], buf.at[pl.ds(8, 8)], sem),
            )

        def start_in(t, buf, sem):
            for cp in in_cps(t, buf, sem):
                cp.start()

        def wait_in(t, buf, sem):
            for cp in in_cps(t, buf, sem):
                cp.wait()

        def w_cp(t, obuf, sem):
            return pltpu.make_async_copy(
                obuf, tbl.at[pl.ds(jj(t) * 128, 128)], sem
            )

        start_in(0, in_a, sg_a)
        wait_in(0, in_a, sg_a)
        start_in(1, in_b, sg_b)
        _transpose_pair(in_a, out_a, iota)
        w_cp(0, out_a, sw_a).start()
        wait_in(1, in_b, sg_b)
        start_in(2, in_a, sg_a)
        _transpose_pair(in_b, out_b, iota)
        w_cp(1, out_b, sw_b).start()

        def body(m, carry):
            t0 = 2 * m
            wait_in(t0, in_a, sg_a)
            start_in(t0 + 1, in_b, sg_b)
            w_cp(t0 - 2, out_a, sw_a).wait()
            _transpose_pair(in_a, out_a, iota)
            w_cp(t0, out_a, sw_a).start()
            wait_in(t0 + 1, in_b, sg_b)
            start_in(t0 + 2, in_a, sg_a)
            w_cp(t0 - 1, out_b, sw_b).wait()
            _transpose_pair(in_b, out_b, iota)
            w_cp(t0 + 1, out_b, sw_b).start()
            return carry

        lax.fori_loop(1, PAIRS_W // 2, body, 0)
        wait_in(PAIRS_W, in_a, sg_a)
        w_cp(PAIRS_W - 2, out_a, sw_a).wait()
        w_cp(PAIRS_W - 1, out_b, sw_b).wait()

    return k


def _gather():
    @functools.partial(
        pl.kernel,
        out_type=jax.ShapeDtypeStruct(
            (NUM_FIELDS, 2, BATCH // 128, 8, 128), jnp.float32
        ),
        mesh=plsc.VectorSubcoreMesh(**_MESH),
        compiler_params=_PARAMS,
        scratch_types=[
            pltpu.VMEM((GROUPS_W, 128), jnp.int32),
            pltpu.VMEM((128, EMBED_DIM), jnp.float32),
            pltpu.VMEM((128, EMBED_DIM), jnp.float32),
            pltpu.VMEM((2, 8, 128), jnp.float32),
            pltpu.VMEM((2, 8, 128), jnp.float32),
            pltpu.SemaphoreType.DMA,
            pltpu.SemaphoreType.DMA,
            pltpu.SemaphoreType.DMA,
            pltpu.SemaphoreType.DMA,
        ],
    )
    def k(x2, tbl, out5, idx_v, buf_a, buf_b, o_a, o_b, sg_a, sg_b, sw_a, sw_b):
        wid = lax.axis_index("s") * NC + lax.axis_index("c")
        row0 = wid * GROUPS_W
        base_e = wid * PER_W
        iota = lax.iota(jnp.int32, LANES)

        pltpu.sync_copy(x2.at[pl.ds(row0, GROUPS_W)], idx_v)

        def addoff(g, carry):
            f = (base_e + g * 128) >> 14
            off = f * FIELD_DIM
            for m in range(8):
                s = pl.ds(m * LANES, LANES)
                idx_v[g, s] = idx_v[g, s] + off
            return carry

        lax.fori_loop(0, GROUPS_W, addoff, 0)

        def gq(g):
            return jnp.minimum(g, GROUPS_W - 1)

        def g_cp(g, buf, sem):
            return pltpu.make_async_copy(tbl.at[idx_v.at[gq(g)]], buf, sem)

        def transpose_block(buf, obuf):
            for c in range(EMBED_DIM):
                ccol = jnp.full((LANES,), c, jnp.int32)
                for m in range(8):
                    rows = iota + m * LANES
                    v = plsc.load_gather(buf, [rows, ccol])
                    obuf[c // 8, c % 8, pl.ds(m * LANES, LANES)] = v

        def w_cps(g, obuf, sem):
            e0 = base_e + g * 128
            f = e0 >> 14
            jb = (e0 & (BATCH - 1)) >> 7
            return (
                pltpu.make_async_copy(obuf.at[0], out5.at[f, 0, jb], sem),
                pltpu.make_async_copy(obuf.at[1], out5.at[f, 1, jb], sem),
            )

        def start_w(g, obuf, sem):
            for cp in w_cps(g, obuf, sem):
                cp.start()

        def wait_w(g, obuf, sem):
            for cp in w_cps(g, obuf, sem):
                cp.wait()

        g_cp(0, buf_a, sg_a).start()
        g_cp(0, buf_a, sg_a).wait()
        g_cp(1, buf_b, sg_b).start()
        transpose_block(buf_a, o_a)
        start_w(0, o_a, sw_a)
        g_cp(1, buf_b, sg_b).wait()
        g_cp(2, buf_a, sg_a).start()
        transpose_block(buf_b, o_b)
        start_w(1, o_b, sw_b)

        def body(m, carry):
            g0 = 2 * m
            g_cp(g0, buf_a, sg_a).wait()
            g_cp(g0 + 1, buf_b, sg_b).start()
            wait_w(g0 - 2, o_a, sw_a)
            transpose_block(buf_a, o_a)
            start_w(g0, o_a, sw_a)
            g_cp(g0 + 1, buf_b, sg_b).wait()
            g_cp(g0 + 2, buf_a, sg_a).start()
            wait_w(g0 - 1, o_b, sw_b)
            transpose_block(buf_b, o_b)
            start_w(g0 + 1, o_b, sw_b)
            return carry

        lax.fori_loop(1, GROUPS_W // 2, body, 0)
        g_cp(GROUPS_W, buf_a, sg_a).wait()
        wait_w(GROUPS_W - 2, o_a, sw_a)
        wait_w(GROUPS_W - 1, o_b, sw_b)

    return k


def kernel(x, table):
    x2 = x.T.reshape(TOTAL // 128, 128)
    t5 = table.T.reshape(2, 8, NPAIR, 128).transpose(0, 2, 1, 3)
    tbl_rm = _convert_table()(t5)
    out5 = _gather()(x2, tbl_rm)
    return out5.transpose(2, 4, 0, 1, 3).reshape(BATCH, NUM_FIELDS, EMBED_DIM)


# speedup vs baseline: 1.7821x; 1.7821x over previous
import functools

import jax
import jax.numpy as jnp
from jax import lax
from jax.experimental import pallas as pl
from jax.experimental.pallas import tpu as pltpu
from jax.experimental.pallas import tpu_sc as plsc

BATCH = 16384
NUM_FIELDS = 26
EMBED_DIM = 16
FIELD_DIM = 40000
TOTAL = BATCH * NUM_FIELDS
SUM_FIELD = NUM_FIELDS * FIELD_DIM

NC, NS, LANES = 2, 16, 16
NW = NC * NS

NPAIR = SUM_FIELD // 128
PAIRS_W = -(-NPAIR // NW)

PER_W = TOTAL // NW
GROUPS_W = PER_W // 128

OUT5_SHAPE = (NUM_FIELDS, 2, BATCH // 128, 8, 128)

_MESH = dict(core_axis_name="c", subcore_axis_name="s")
_PARAMS = pltpu.CompilerParams(
    use_tc_tiling_on_sc=False,
    needs_layout_passes=False,
    disable_bounds_checks=True,
)


def _convert_table():
    @functools.partial(
        pl.kernel,
        out_type=jax.ShapeDtypeStruct((SUM_FIELD * EMBED_DIM,), jnp.float32),
        mesh=plsc.VectorSubcoreMesh(**_MESH),
        compiler_params=_PARAMS,
        scratch_types=[
            pltpu.VMEM((16, 128), jnp.float32),
            pltpu.VMEM((16, 128), jnp.float32),
            pltpu.VMEM((128 * EMBED_DIM,), jnp.float32),
            pltpu.VMEM((128 * EMBED_DIM,), jnp.float32),
            pltpu.SemaphoreType.DMA,
            pltpu.SemaphoreType.DMA,
            pltpu.SemaphoreType.DMA,
            pltpu.SemaphoreType.DMA,
        ],
    )
    def k(t5, tbl, in_a, in_b, out_a, out_b, sg_a, sg_b, sw_a, sw_b):
        wid = lax.axis_index("s") * NC + lax.axis_index("c")
        start = wid * PAIRS_W
        iota = lax.iota(jnp.int32, LANES)

        def jj(t):
            return jnp.minimum(start + t, NPAIR - 1)

        def in_cps(t, buf, sem):
            j = jj(t)
            return (
                pltpu.make_async_copy(t5.at[0, j], buf.at[pl.ds(0, 8)], sem),
                pltpu.make_async_copy(t5.at[1, j], buf.at[pl.ds(8, 8)], sem),
            )

        def start_in(t, buf, sem):
            for cp in in_cps(t, buf, sem):
                cp.start()

        def wait_in(t, buf, sem):
            for cp in in_cps(t, buf, sem):
                cp.wait()

        def transpose_pair(src, dst):
            @functools.partial(plsc.parallel_loop, 0, 128, unroll=8)
            def _(b):
                col = jnp.full((LANES,), b, jnp.int32)
                v = plsc.load_gather(src, [iota, col])
                dst[pl.ds(b * EMBED_DIM, LANES)] = v

        def w_cp(t, obuf, sem):
            return pltpu.make_async_copy(
                obuf, tbl.at[pl.ds(jj(t) * (128 * EMBED_DIM), 128 * EMBED_DIM)], sem
            )

        start_in(0, in_a, sg_a)
        wait_in(0, in_a, sg_a)
        start_in(1, in_b, sg_b)
        transpose_pair(in_a, out_a)
        w_cp(0, out_a, sw_a).start()
        wait_in(1, in_b, sg_b)
        start_in(2, in_a, sg_a)
        transpose_pair(in_b, out_b)
        w_cp(1, out_b, sw_b).start()

        def body(m, carry):
            t0 = 2 * m
            wait_in(t0, in_a, sg_a)
            start_in(t0 + 1, in_b, sg_b)
            w_cp(t0 - 2, out_a, sw_a).wait()
            transpose_pair(in_a, out_a)
            w_cp(t0, out_a, sw_a).start()
            wait_in(t0 + 1, in_b, sg_b)
            start_in(t0 + 2, in_a, sg_a)
            w_cp(t0 - 1, out_b, sw_b).wait()
            transpose_pair(in_b, out_b)
            w_cp(t0 + 1, out_b, sw_b).start()
            return carry

        lax.fori_loop(1, PAIRS_W // 2, body, 0)
        wait_in(PAIRS_W, in_a, sg_a)
        w_cp(PAIRS_W - 2, out_a, sw_a).wait()
        w_cp(PAIRS_W - 1, out_b, sw_b).wait()

    return k


def _gather():
    @functools.partial(
        pl.kernel,
        out_type=jax.ShapeDtypeStruct(
            (NUM_FIELDS * 2 * (BATCH // 128) * 8 * 128,), jnp.float32
        ),
        mesh=plsc.VectorSubcoreMesh(**_MESH),
        compiler_params=_PARAMS,
        scratch_types=[
            pltpu.VMEM((GROUPS_W, 128), jnp.int32),
            pltpu.VMEM((128, EMBED_DIM), jnp.float32),
            pltpu.VMEM((128, EMBED_DIM), jnp.float32),
            pltpu.VMEM((128 * EMBED_DIM,), jnp.float32),
            pltpu.VMEM((128 * EMBED_DIM,), jnp.float32),
            pltpu.SemaphoreType.DMA,
            pltpu.SemaphoreType.DMA,
            pltpu.SemaphoreType.DMA,
            pltpu.SemaphoreType.DMA,
        ],
    )
    def k(x2, tbl, out5, idx_v, buf_a, buf_b, o_a, o_b, sg_a, sg_b, sw_a, sw_b):
        wid = lax.axis_index("s") * NC + lax.axis_index("c")
        row0 = wid * GROUPS_W
        base_e = wid * PER_W
        iota = lax.iota(jnp.int32, LANES)

        pltpu.sync_copy(x2.at[pl.ds(row0, GROUPS_W)], idx_v)

        def addoff(g, carry):
            f = (base_e + g * 128) >> 14
            off = f * FIELD_DIM
            for m in range(8):
                s = pl.ds(m * LANES, LANES)
                idx_v[g, s] = idx_v[g, s] + off
            return carry

        lax.fori_loop(0, GROUPS_W, addoff, 0)

        def gq(g):
            return jnp.minimum(g, GROUPS_W - 1)

        def g_cp(g, buf, sem):
            return pltpu.make_async_copy(tbl.at[idx_v.at[gq(g)]], buf, sem)

        def transpose_block(buf, obuf):
            @functools.partial(plsc.parallel_loop, 0, 128, unroll=8)
            def _(t):
                c = t // 8
                m = t % 8
                ccol = jnp.full((LANES,), c, jnp.int32)
                rows = iota + m * LANES
                v = plsc.load_gather(buf, [rows, ccol])
                obuf[pl.ds(c * 128 + m * LANES, LANES)] = v

        def w_cps(g, obuf, sem):
            e0 = base_e + g * 128
            f = e0 >> 14
            jb = (e0 & (BATCH - 1)) >> 7
            base0 = ((f * 2 + 0) * (BATCH // 128) + jb) * 1024
            base1 = ((f * 2 + 1) * (BATCH // 128) + jb) * 1024
            return (
                pltpu.make_async_copy(
                    obuf.at[pl.ds(0, 1024)], out5.at[pl.ds(base0, 1024)], sem
                ),
                pltpu.make_async_copy(
                    obuf.at[pl.ds(1024, 1024)], out5.at[pl.ds(base1, 1024)], sem
                ),
            )

        def start_w(g, obuf, sem):
            for cp in w_cps(g, obuf, sem):
                cp.start()

        def wait_w(g, obuf, sem):
            for cp in w_cps(g, obuf, sem):
                cp.wait()

        g_cp(0, buf_a, sg_a).start()
        g_cp(0, buf_a, sg_a).wait()
        g_cp(1, buf_b, sg_b).start()
        transpose_block(buf_a, o_a)
        start_w(0, o_a, sw_a)
        g_cp(1, buf_b, sg_b).wait()
        g_cp(2, buf_a, sg_a).start()
        transpose_block(buf_b, o_b)
        start_w(1, o_b, sw_b)

        def body(m, carry):
            g0 = 2 * m
            g_cp(g0, buf_a, sg_a).wait()
            g_cp(g0 + 1, buf_b, sg_b).start()
            wait_w(g0 - 2, o_a, sw_a)
            transpose_block(buf_a, o_a)
            start_w(g0, o_a, sw_a)
            g_cp(g0 + 1, buf_b, sg_b).wait()
            g_cp(g0 + 2, buf_a, sg_a).start()
            wait_w(g0 - 1, o_b, sw_b)
            transpose_block(buf_b, o_b)
            start_w(g0 + 1, o_b, sw_b)
            return carry

        lax.fori_loop(1, GROUPS_W // 2, body, 0)
        g_cp(GROUPS_W, buf_a, sg_a).wait()
        wait_w(GROUPS_W - 2, o_a, sw_a)
        wait_w(GROUPS_W - 1, o_b, sw_b)

    return k


def kernel(x, table):
    x2 = x.T.reshape(TOTAL // 128, 128)
    t5 = table.T.reshape(2, 8, NPAIR, 128).transpose(0, 2, 1, 3)
    tbl_rm = _convert_table()(t5).reshape(SUM_FIELD, EMBED_DIM)
    out5 = _gather()(x2, tbl_rm).reshape(OUT5_SHAPE)
    return out5.transpose(2, 4, 0, 1, 3).reshape(BATCH, NUM_FIELDS, EMBED_DIM)


# speedup vs baseline: 1.7867x; 1.0025x over previous
import functools

import jax
import jax.numpy as jnp
from jax import lax
from jax.experimental import pallas as pl
from jax.experimental.pallas import tpu as pltpu
from jax.experimental.pallas import tpu_sc as plsc

BATCH = 16384
NUM_FIELDS = 26
EMBED_DIM = 16
FIELD_DIM = 40000
TOTAL = BATCH * NUM_FIELDS
SUM_FIELD = NUM_FIELDS * FIELD_DIM

NC, NS, LANES = 2, 16, 16
NW = NC * NS

NPAIR = SUM_FIELD // 128
PAIRS_W = -(-NPAIR // NW)

PER_W = TOTAL // NW
GROUPS_W = PER_W // 128

OUT5_SHAPE = (NUM_FIELDS, 2, BATCH // 128, 8, 128)

_MESH = dict(core_axis_name="c", subcore_axis_name="s")
_PARAMS = pltpu.CompilerParams(
    use_tc_tiling_on_sc=False,
    needs_layout_passes=False,
    disable_bounds_checks=True,
)


def _convert_table():
    @functools.partial(
        pl.kernel,
        out_type=jax.ShapeDtypeStruct((SUM_FIELD * EMBED_DIM,), jnp.float32),
        mesh=plsc.VectorSubcoreMesh(**_MESH),
        compiler_params=_PARAMS,
        scratch_types=[
            pltpu.VMEM((16, 128), jnp.float32),
            pltpu.VMEM((16, 128), jnp.float32),
            pltpu.VMEM((128 * EMBED_DIM,), jnp.float32),
            pltpu.VMEM((128 * EMBED_DIM,), jnp.float32),
            pltpu.SemaphoreType.DMA,
            pltpu.SemaphoreType.DMA,
            pltpu.SemaphoreType.DMA,
            pltpu.SemaphoreType.DMA,
        ],
    )
    def k(t5, tbl, in_a, in_b, out_a, out_b, sg_a, sg_b, sw_a, sw_b):
        wid = lax.axis_index("s") * NC + lax.axis_index("c")
        start = wid * PAIRS_W
        iota = lax.iota(jnp.int32, LANES)

        def jj(t):
            return jnp.minimum(start + t, NPAIR - 1)

        def in_cps(t, buf, sem):
            j = jj(t)
            return (
                pltpu.make_async_copy(t5.at[0, j], buf.at[pl.ds(0, 8)], sem),
                pltpu.make_async_copy(t5.at[1, j], buf.at[pl.ds(8, 8)], sem),
            )

        def start_in(t, buf, sem):
            for cp in in_cps(t, buf, sem):
                cp.start()

        def wait_in(t, buf, sem):
            for cp in in_cps(t, buf, sem):
                cp.wait()

        def transpose_pair(src, dst):
            @functools.partial(plsc.parallel_loop, 0, 128, unroll=16)
            def _(b):
                col = jnp.full((LANES,), b, jnp.int32)
                v = plsc.load_gather(src, [iota, col])
                dst[pl.ds(b * EMBED_DIM, LANES)] = v

        def w_cp(t, obuf, sem):
            return pltpu.make_async_copy(
                obuf, tbl.at[pl.ds(jj(t) * (128 * EMBED_DIM), 128 * EMBED_DIM)], sem
            )

        start_in(0, in_a, sg_a)
        wait_in(0, in_a, sg_a)
        start_in(1, in_b, sg_b)
        transpose_pair(in_a, out_a)
        w_cp(0, out_a, sw_a).start()
        wait_in(1, in_b, sg_b)
        start_in(2, in_a, sg_a)
        transpose_pair(in_b, out_b)
        w_cp(1, out_b, sw_b).start()

        def body(m, carry):
            t0 = 2 * m
            wait_in(t0, in_a, sg_a)
            start_in(t0 + 1, in_b, sg_b)
            w_cp(t0 - 2, out_a, sw_a).wait()
            transpose_pair(in_a, out_a)
            w_cp(t0, out_a, sw_a).start()
            wait_in(t0 + 1, in_b, sg_b)
            start_in(t0 + 2, in_a, sg_a)
            w_cp(t0 - 1, out_b, sw_b).wait()
            transpose_pair(in_b, out_b)
            w_cp(t0 + 1, out_b, sw_b).start()
            return carry

        lax.fori_loop(1, PAIRS_W // 2, body, 0)
        wait_in(PAIRS_W, in_a, sg_a)
        w_cp(PAIRS_W - 2, out_a, sw_a).wait()
        w_cp(PAIRS_W - 1, out_b, sw_b).wait()

    return k


def _gather():
    @functools.partial(
        pl.kernel,
        out_type=jax.ShapeDtypeStruct(
            (NUM_FIELDS * 2 * (BATCH // 128) * 8 * 128,), jnp.float32
        ),
        mesh=plsc.VectorSubcoreMesh(**_MESH),
        compiler_params=_PARAMS,
        scratch_types=[
            pltpu.VMEM((GROUPS_W, 128), jnp.int32),
            pltpu.VMEM((128, EMBED_DIM), jnp.float32),
            pltpu.VMEM((128, EMBED_DIM), jnp.float32),
            pltpu.VMEM((128 * EMBED_DIM,), jnp.float32),
            pltpu.VMEM((128 * EMBED_DIM,), jnp.float32),
            pltpu.SemaphoreType.DMA,
            pltpu.SemaphoreType.DMA,
            pltpu.SemaphoreType.DMA,
            pltpu.SemaphoreType.DMA,
        ],
    )
    def k(x2, tbl, out5, idx_v, buf_a, buf_b, o_a, o_b, sg_a, sg_b, sw_a, sw_b):
        wid = lax.axis_index("s") * NC + lax.axis_index("c")
        row0 = wid * GROUPS_W
        base_e = wid * PER_W
        iota = lax.iota(jnp.int32, LANES)

        pltpu.sync_copy(x2.at[pl.ds(row0, GROUPS_W)], idx_v)

        def addoff(g, carry):
            f = (base_e + g * 128) >> 14
            off = f * FIELD_DIM
            for m in range(8):
                s = pl.ds(m * LANES, LANES)
                idx_v[g, s] = idx_v[g, s] + off
            return carry

        lax.fori_loop(0, GROUPS_W, addoff, 0)

        def gq(g):
            return jnp.minimum(g, GROUPS_W - 1)

        def g_cp(g, buf, sem):
            return pltpu.make_async_copy(tbl.at[idx_v.at[gq(g)]], buf, sem)

        def transpose_block(buf, obuf):
            @functools.partial(plsc.parallel_loop, 0, 128, unroll=16)
            def _(t):
                c = t // 8
                m = t % 8
                ccol = jnp.full((LANES,), c, jnp.int32)
                rows = iota + m * LANES
                v = plsc.load_gather(buf, [rows, ccol])
                obuf[pl.ds(c * 128 + m * LANES, LANES)] = v

        def w_cps(g, obuf, sem):
            e0 = base_e + g * 128
            f = e0 >> 14
            jb = (e0 & (BATCH - 1)) >> 7
            base0 = ((f * 2 + 0) * (BATCH // 128) + jb) * 1024
            base1 = ((f * 2 + 1) * (BATCH // 128) + jb) * 1024
            return (
                pltpu.make_async_copy(
                    obuf.at[pl.ds(0, 1024)], out5.at[pl.ds(base0, 1024)], sem
                ),
                pltpu.make_async_copy(
                    obuf.at[pl.ds(1024, 1024)], out5.at[pl.ds(base1, 1024)], sem
                ),
            )

        def start_w(g, obuf, sem):
            for cp in w_cps(g, obuf, sem):
                cp.start()

        def wait_w(g, obuf, sem):
            for cp in w_cps(g, obuf, sem):
                cp.wait()

        g_cp(0, buf_a, sg_a).start()
        g_cp(0, buf_a, sg_a).wait()
        g_cp(1, buf_b, sg_b).start()
        transpose_block(buf_a, o_a)
        start_w(0, o_a, sw_a)
        g_cp(1, buf_b, sg_b).wait()
        g_cp(2, buf_a, sg_a).start()
        transpose_block(buf_b, o_b)
        start_w(1, o_b, sw_b)

        def body(m, carry):
            g0 = 2 * m
            g_cp(g0, buf_a, sg_a).wait()
            g_cp(g0 + 1, buf_b, sg_b).start()
            wait_w(g0 - 2, o_a, sw_a)
            transpose_block(buf_a, o_a)
            start_w(g0, o_a, sw_a)
            g_cp(g0 + 1, buf_b, sg_b).wait()
            g_cp(g0 + 2, buf_a, sg_a).start()
            wait_w(g0 - 1, o_b, sw_b)
            transpose_block(buf_b, o_b)
            start_w(g0 + 1, o_b, sw_b)
            return carry

        lax.fori_loop(1, GROUPS_W // 2, body, 0)
        g_cp(GROUPS_W, buf_a, sg_a).wait()
        wait_w(GROUPS_W - 2, o_a, sw_a)
        wait_w(GROUPS_W - 1, o_b, sw_b)

    return k


def kernel(x, table):
    x2 = x.T.reshape(TOTAL // 128, 128)
    t5 = table.T.reshape(2, 8, NPAIR, 128).transpose(0, 2, 1, 3)
    tbl_rm = _convert_table()(t5).reshape(SUM_FIELD, EMBED_DIM)
    out5 = _gather()(x2, tbl_rm).reshape(OUT5_SHAPE)
    return out5.transpose(2, 4, 0, 1, 3).reshape(BATCH, NUM_FIELDS, EMBED_DIM)
